# baseline (speedup 1.0000x reference)
import jax
import jax.numpy as jnp
from jax.experimental import pallas as pl
from jax.experimental.pallas import tpu as pltpu

NE = 1024
D = 64
B = 32
T = 1024
N = B * T


def _vq_body(x_ref, w_ref, q_ref, idx_ref, loss_ref, perp_ref,
             wt, riota, counts, acc):
    b = pl.program_id(0)

    @pl.when(b == 0)
    def _init():
        counts[...] = jnp.zeros_like(counts)
        acc[0, 0] = 0.0
        wt[...] = w_ref[...].T
        riota[...] = jax.lax.broadcasted_iota(
            jnp.int32, (NE, T), 0).astype(jnp.float32)

    f_cb = x_ref[0]
    w = w_ref[...]
    wsq = jnp.sum(w * w, axis=1, keepdims=True)
    fsq = jnp.sum(f_cb * f_cb, axis=0, keepdims=True)
    mm = jax.lax.dot_general(
        w, f_cb, (((1,), (0,)), ((), ())),
        preferred_element_type=jnp.float32)
    dist = (fsq + wsq) - 2.0 * mm

    m = jnp.min(dist, axis=0, keepdims=True)
    row = riota[...]
    idxf = jnp.min(jnp.where(dist == m, row, jnp.float32(NE)),
                   axis=0, keepdims=True)
    oh = jnp.where(row == idxf, jnp.float32(1.0), jnp.float32(0.0))

    q = jax.lax.dot_general(
        wt[...], oh, (((1,), (0,)), ((), ())),
        preferred_element_type=jnp.float32)
    cnt = jax.lax.dot_general(
        oh, jnp.ones((T, 1), jnp.float32), (((1,), (0,)), ((), ())),
        preferred_element_type=jnp.float32)

    acc[0, 0] += jnp.sum(m)
    counts[...] += cnt

    q_ref[0] = q
    idx_ref[0, 0] = idxf[0].astype(jnp.int32)

    @pl.when(b == B - 1)
    def _fin():
        loss_ref[0, 0] = 0.25 * acc[0, 0] / (N * D)
        p = counts[...] / N
        perp_ref[0, 0] = jnp.exp(-jnp.sum(p * jnp.log(p + 1e-10)))


def kernel(x, W):
    x4 = x.reshape(B, D, T)
    q4, idx3, loss, perp = pl.pallas_call(
        _vq_body,
        grid=(B,),
        in_specs=[
            pl.BlockSpec((1, D, T), lambda b: (b, 0, 0)),
            pl.BlockSpec((NE, D), lambda b: (0, 0)),
        ],
        out_specs=(
            pl.BlockSpec((1, D, T), lambda b: (b, 0, 0)),
            pl.BlockSpec((1, 1, T), lambda b: (b, 0, 0)),
            pl.BlockSpec(memory_space=pltpu.SMEM),
            pl.BlockSpec(memory_space=pltpu.SMEM),
        ),
        out_shape=(
            jax.ShapeDtypeStruct((B, D, T), jnp.float32),
            jax.ShapeDtypeStruct((B, 1, T), jnp.int32),
            jax.ShapeDtypeStruct((1, 1), jnp.float32),
            jax.ShapeDtypeStruct((1, 1), jnp.float32),
        ),
        scratch_shapes=[
            pltpu.VMEM((D, NE), jnp.float32),
            pltpu.VMEM((NE, T), jnp.float32),
            pltpu.VMEM((NE, 1), jnp.float32),
            pltpu.SMEM((1, 1), jnp.float32),
        ],
    )(x4, W)
    quantized = q4.reshape(32, 64, 32, 32)
    indices = idx3.reshape(32, 32, 32)
    return quantized, loss[0, 0], indices, perp[0, 0]


# speedup vs baseline: 1.1374x; 1.1374x over previous
import jax
import jax.numpy as jnp
from jax.experimental import pallas as pl
from jax.experimental.pallas import tpu as pltpu

NE = 1024
D = 64
B = 32
T = 1024
N = B * T


def _vq_body(x_ref, w_ref, q_ref, idx_ref, loss_ref, perp_ref,
             wt, w2, counts, acc):
    s = pl.program_id(0)

    @pl.when(s == 0)
    def _init():
        w = w_ref[...]
        counts[...] = jnp.zeros_like(counts)
        acc[0, 0] = 0.0
        wt[...] = w.T
        w2[...] = w * jnp.float32(-2.0)

    row = jax.lax.broadcasted_iota(jnp.int32, (NE, T), 0)

    f_cb = x_ref[0]
    wsq = jnp.sum(w_ref[...] * w_ref[...], axis=1, keepdims=True)
    fsq = jnp.sum(f_cb * f_cb, axis=0, keepdims=True)
    mm2 = jax.lax.dot_general(
        w2[...], f_cb, (((1,), (0,)), ((), ())),
        preferred_element_type=jnp.float32)
    dist = (fsq + wsq) + mm2
    m = jnp.min(dist, axis=0, keepdims=True)
    idx = jnp.min(jnp.where(dist == m, row, NE),
                  axis=0, keepdims=True)
    oh = jnp.where(row == idx, jnp.float32(1.0), jnp.float32(0.0))
    q = jax.lax.dot_general(
        wt[...], oh, (((1,), (0,)), ((), ())),
        preferred_element_type=jnp.float32)

    acc[0, 0] += jnp.sum(m)
    counts[...] += jnp.sum(oh, axis=1, keepdims=True)
    q_ref[0] = q
    idx_ref[0, 0] = idx[0]

    @pl.when(s == B - 1)
    def _fin():
        loss_ref[0, 0] = 0.25 * acc[0, 0] / (N * D)
        p = counts[...] / N
        perp_ref[0, 0] = jnp.exp(-jnp.sum(p * jnp.log(p + 1e-10)))


def kernel(x, W):
    x4 = x.reshape(B, D, T)
    q4, idx3, loss, perp = pl.pallas_call(
        _vq_body,
        grid=(B,),
        in_specs=[
            pl.BlockSpec((1, D, T), lambda s: (s, 0, 0)),
            pl.BlockSpec((NE, D), lambda s: (0, 0)),
        ],
        out_specs=(
            pl.BlockSpec((1, D, T), lambda s: (s, 0, 0)),
            pl.BlockSpec((1, 1, T), lambda s: (s, 0, 0)),
            pl.BlockSpec(memory_space=pltpu.SMEM),
            pl.BlockSpec(memory_space=pltpu.SMEM),
        ),
        out_shape=(
            jax.ShapeDtypeStruct((B, D, T), jnp.float32),
            jax.ShapeDtypeStruct((B, 1, T), jnp.int32),
            jax.ShapeDtypeStruct((1, 1), jnp.float32),
            jax.ShapeDtypeStruct((1, 1), jnp.float32),
        ),
        scratch_shapes=[
            pltpu.VMEM((D, NE), jnp.float32),
            pltpu.VMEM((NE, D), jnp.float32),
            pltpu.VMEM((NE, 1), jnp.float32),
            pltpu.SMEM((1, 1), jnp.float32),
        ],
    )(x4, W)
    quantized = q4.reshape(32, 64, 32, 32)
    indices = idx3.reshape(32, 32, 32)
    return quantized, loss[0, 0], indices, perp[0, 0]
